# Initial kernel scaffold; baseline (speedup 1.0000x reference)
import jax
import jax.numpy as jnp
from jax.experimental import pallas as pl


def kernel(features, structural_indices, n_structures, W):
    raise NotImplementedError("write your pallas kernel here")


# speedup vs baseline: 4.9602x; 4.9602x over previous
import functools

import jax
import jax.numpy as jnp
from jax import lax
from jax.experimental import pallas as pl
from jax.experimental.pallas import tpu as pltpu
from jax.experimental.pallas import tpu_sc as plsc

_N = 320000
_D = 128
_S = 10000
_B = 128
_NC = 2
_NS = 16
_NW = _NC * _NS
_NB = _N // _B
_ROWS_PER_TILE = 624
_TAIL_ROWS = _S - _NS * _ROWS_PER_TILE


def _sc_segment_sum(features, sids2d):
    mesh = plsc.VectorSubcoreMesh(core_axis_name="c", subcore_axis_name="s")

    @functools.partial(
        pl.kernel,
        out_type=jax.ShapeDtypeStruct((_NC, _S, _D), jnp.float32),
        mesh=mesh,
        scratch_types=[
            pltpu.VMEM_SHARED((_S, _D), jnp.float32),
            pltpu.VMEM((_B, _D), jnp.float32),
            pltpu.VMEM((1, _B), jnp.int32),
        ],
    )
    def seg_kernel(feat_hbm, idx_hbm, out_hbm, acc_sh, fbuf, ibuf):
        c = lax.axis_index("c")
        s = lax.axis_index("s")
        wid = c * _NS + s

        zvec = jnp.zeros((16,), jnp.float32)

        def _zero_row(i, _):
            for j in range(_D // 16):
                fbuf[i, pl.ds(j * 16, 16)] = zvec
            return 0

        lax.fori_loop(0, _B, _zero_row, 0)

        row0 = s * _ROWS_PER_TILE
        nfull = _ROWS_PER_TILE // _B
        rem = _ROWS_PER_TILE - nfull * _B
        for k in range(nfull):
            pltpu.sync_copy(fbuf, acc_sh.at[pl.ds(row0 + k * _B, _B)])
        if rem:
            pltpu.sync_copy(fbuf.at[pl.ds(0, rem)],
                            acc_sh.at[pl.ds(row0 + nfull * _B, rem)])

        @pl.when(s == _NS - 1)
        def _zero_tail():
            pltpu.sync_copy(fbuf.at[pl.ds(0, _TAIL_ROWS)],
                            acc_sh.at[pl.ds(_NS * _ROWS_PER_TILE, _TAIL_ROWS)])

        plsc.subcore_barrier()

        nsteps = (_NB + _NW - 1) // _NW

        def _step(j, _):
            batch = j * _NW + wid

            @pl.when(batch < _NB)
            def _():
                pltpu.sync_copy(feat_hbm.at[pl.ds(batch * _B, _B)], fbuf)
                pltpu.sync_copy(idx_hbm.at[batch], ibuf.at[0])
                pltpu.sync_copy(fbuf, acc_sh.at[ibuf.at[0]], add=True)

            return 0

        lax.fori_loop(0, nsteps, _step, 0)
        plsc.subcore_barrier()

        for k in range(nfull):
            pltpu.sync_copy(acc_sh.at[pl.ds(row0 + k * _B, _B)],
                            out_hbm.at[c, pl.ds(row0 + k * _B, _B)])
        if rem:
            pltpu.sync_copy(acc_sh.at[pl.ds(row0 + nfull * _B, rem)],
                            out_hbm.at[c, pl.ds(row0 + nfull * _B, rem)])

        @pl.when(s == _NS - 1)
        def _copy_tail():
            pltpu.sync_copy(acc_sh.at[pl.ds(_NS * _ROWS_PER_TILE, _TAIL_ROWS)],
                            out_hbm.at[c, pl.ds(_NS * _ROWS_PER_TILE,
                                                _TAIL_ROWS)])

    return seg_kernel(features, sids2d)


def _mm_body(p_ref, w_ref, o_ref):
    o_ref[...] = jnp.dot(p_ref[0] + p_ref[1], w_ref[...],
                         preferred_element_type=jnp.float32)


_mm = pl.pallas_call(
    _mm_body,
    grid=(10,),
    in_specs=[
        pl.BlockSpec((_NC, _S // 10, _D), lambda i: (0, i, 0)),
        pl.BlockSpec((_D, _D), lambda i: (0, 0)),
    ],
    out_specs=pl.BlockSpec((_S // 10, _D), lambda i: (i, 0)),
    out_shape=jax.ShapeDtypeStruct((_S, _D), jnp.float32),
)


def kernel(features, structural_indices, n_structures, W):
    del n_structures
    sids2d = structural_indices.astype(jnp.int32).reshape(_NB, _B)
    partials = _sc_segment_sum(features, sids2d)
    return _mm(partials, W)


# speedup vs baseline: 7.9169x; 1.5961x over previous
import functools

import jax
import jax.numpy as jnp
from jax import lax
from jax.experimental import pallas as pl
from jax.experimental.pallas import tpu as pltpu
from jax.experimental.pallas import tpu_sc as plsc

_N = 320000
_D = 128
_S = 10000
_B = 128
_NC = 2
_NS = 16
_NW = _NC * _NS
_NB = _N // _B
_B_PER_W = (_NB + _NW - 1) // _NW
_B_PER_W = 80
_NRING = 3
_ROWS_PER_TILE = 624
_TAIL_ROWS = _S - _NS * _ROWS_PER_TILE


def _sc_segment_sum(features, sids2d, zrows):
    mesh = plsc.VectorSubcoreMesh(core_axis_name="c", subcore_axis_name="s")

    @functools.partial(
        pl.kernel,
        out_type=jax.ShapeDtypeStruct((_NC, _S, _D), jnp.float32),
        mesh=mesh,
        scratch_types=[
            pltpu.VMEM_SHARED((_S, _D), jnp.float32),
            pltpu.VMEM((_NRING, _B, _D), jnp.float32),
            pltpu.VMEM((_NRING, 1, _B), jnp.int32),
            pltpu.SemaphoreType.DMA((_NRING,)),
            pltpu.SemaphoreType.DMA((_NRING,)),
        ],
    )
    def seg_kernel(feat_hbm, idx_hbm, zero_hbm, out_hbm, acc_sh, fbuf, ibuf,
                   lsem, ssem):
        c = lax.axis_index("c")
        s = lax.axis_index("s")
        wid = c * _NS + s
        cnt = jnp.clip(_NB - wid * _B_PER_W, 0, _B_PER_W)

        row0 = s * _ROWS_PER_TILE
        pltpu.sync_copy(zero_hbm.at[pl.ds(row0, _ROWS_PER_TILE)],
                        acc_sh.at[pl.ds(row0, _ROWS_PER_TILE)])

        @pl.when(s == _NS - 1)
        def _zero_tail():
            pltpu.sync_copy(
                zero_hbm.at[pl.ds(_NS * _ROWS_PER_TILE, _TAIL_ROWS)],
                acc_sh.at[pl.ds(_NS * _ROWS_PER_TILE, _TAIL_ROWS)])

        plsc.subcore_barrier()

        def _load(t, slot):
            g = wid * _B_PER_W + t
            pltpu.async_copy(feat_hbm.at[pl.ds(g * _B, _B)],
                             fbuf.at[slot], lsem.at[slot])
            pltpu.async_copy(idx_hbm.at[g], ibuf.at[slot, 0], lsem.at[slot])

        def _wait_load(slot):
            pltpu.make_async_copy(feat_hbm.at[pl.ds(0, _B)],
                                  fbuf.at[slot], lsem.at[slot]).wait()
            pltpu.make_async_copy(idx_hbm.at[0], ibuf.at[slot, 0],
                                  lsem.at[slot]).wait()

        def _scatter(slot):
            pltpu.async_copy(fbuf.at[slot], acc_sh.at[ibuf.at[slot, 0]],
                             ssem.at[slot], add=True)

        def _wait_scatter(slot):
            pltpu.make_async_copy(fbuf.at[slot],
                                  acc_sh.at[ibuf.at[slot, 0]],
                                  ssem.at[slot]).wait()

        @pl.when(0 < cnt)
        def _():
            _load(0, 0)

        @pl.when(1 < cnt)
        def _():
            _load(1, 1)

        def _step(t, _):
            slot = t % _NRING

            @pl.when(t < cnt)
            def _():
                _wait_load(slot)
                _scatter(slot)

            nslot = (t + 2) % _NRING

            @pl.when((t >= 1) & (t - 1 < cnt))
            def _():
                _wait_scatter(nslot)

            @pl.when(t + 2 < cnt)
            def _():
                _load(t + 2, nslot)

            return 0

        lax.fori_loop(0, _B_PER_W, _step, 0)

        @pl.when(cnt == _B_PER_W)
        def _drain():
            _wait_scatter((_B_PER_W - 1) % _NRING)

        plsc.subcore_barrier()

        pltpu.sync_copy(acc_sh.at[pl.ds(row0, _ROWS_PER_TILE)],
                        out_hbm.at[c, pl.ds(row0, _ROWS_PER_TILE)])

        @pl.when(s == _NS - 1)
        def _copy_tail():
            pltpu.sync_copy(acc_sh.at[pl.ds(_NS * _ROWS_PER_TILE, _TAIL_ROWS)],
                            out_hbm.at[c, pl.ds(_NS * _ROWS_PER_TILE,
                                                _TAIL_ROWS)])

    return seg_kernel(features, sids2d, zrows)


def _mm_body(p_ref, w_ref, o_ref):
    o_ref[...] = jnp.dot(p_ref[0] + p_ref[1], w_ref[...],
                         preferred_element_type=jnp.float32)


_mm = pl.pallas_call(
    _mm_body,
    grid=(10,),
    in_specs=[
        pl.BlockSpec((_NC, _S // 10, _D), lambda i: (0, i, 0)),
        pl.BlockSpec((_D, _D), lambda i: (0, 0)),
    ],
    out_specs=pl.BlockSpec((_S // 10, _D), lambda i: (i, 0)),
    out_shape=jax.ShapeDtypeStruct((_S, _D), jnp.float32),
)


def kernel(features, structural_indices, n_structures, W):
    del n_structures
    sids2d = structural_indices.astype(jnp.int32).reshape(_NB, _B)
    zrows = jnp.zeros((_S, _D), jnp.float32)
    partials = _sc_segment_sum(features, sids2d, zrows)
    return _mm(partials, W)
